# baseline (speedup 1.0000x reference)
import functools

import jax
import jax.numpy as jnp
from jax.experimental import pallas as pl
from jax.experimental.pallas import tpu as pltpu

_B = 512
_K = 3
_R = 2048
_W = 16
_NEG = float("-inf")
_BIGF = 1e9
_INTERPRET = False


def _pool_kernel(w0ref, w1ref, xref, bref, out_ref, cnt_ref, top3_ref,
                 *, nblocks, n, d):
    g = pl.program_id(0)

    @pl.when(g == 0)
    def _init():
        out_ref[...] = jnp.zeros_like(out_ref)
        cnt_ref[...] = jnp.zeros_like(cnt_ref)
        lane = jax.lax.broadcasted_iota(jnp.int32, (_B, 128), 1)
        top3_ref[...] = jnp.where(lane < _K, _NEG, _BIGF)

    xb = xref[...]
    rowid = g * _R + jax.lax.broadcasted_iota(jnp.int32, (_R, 1), 0)
    xb = jnp.where(rowid < n, xb, 0.0)
    xbf = xb.astype(jnp.bfloat16)
    bt = bref[g]

    kj = jnp.transpose(xb[:, -1:])
    idxf = (g * _R + jax.lax.broadcasted_iota(jnp.int32, (1, _R), 1)
            ).astype(jnp.float32)
    w0 = w0ref[g]
    w1 = w1ref[g]
    wseg = jax.lax.broadcasted_iota(jnp.int32, (_W, 1), 0)
    lane8 = jax.lax.broadcasted_iota(jnp.int32, (_W, 8), 1)
    blo = jnp.float32(g) * _R

    def wbody(w, carry):
        act = bt == (w * _W + wseg)
        out_ref[pl.ds(w * _W, _W), d:2 * d] += jnp.dot(
            act.astype(jnp.bfloat16), xbf,
            preferred_element_type=jnp.float32)
        cnt_ref[pl.ds(w * _W, _W), 0:1] += jnp.sum(
            act.astype(jnp.float32), axis=1, keepdims=True)

        loc = []
        for _ in range(_K):
            ck = jnp.where(act, kj, _NEG)
            m = jnp.max(ck, axis=1, keepdims=True)
            eqm = act & (kj == m)
            im = jnp.min(jnp.where(eqm, idxf, _BIGF), axis=1, keepdims=True)
            act = act & ~(eqm & (idxf == im))
            loc.append((m, im))

        prev = top3_ref[pl.ds(w * _W, _W), :]
        prevk = [prev[:, t:t + 1] for t in range(_K)]
        previ = [prev[:, _K + t:_K + t + 1] for t in range(_K)]
        ck8 = jnp.full((_W, 8), _NEG, jnp.float32)
        ci8 = jnp.full((_W, 8), _BIGF, jnp.float32)
        for t in range(_K):
            ck8 = jnp.where(lane8 == t, loc[t][0], ck8)
            ci8 = jnp.where(lane8 == t, loc[t][1], ci8)
            ck8 = jnp.where(lane8 == _K + t, prevk[t], ck8)
            ci8 = jnp.where(lane8 == _K + t, previ[t], ci8)
        out3 = jnp.full((_W, 128), _BIGF, jnp.float32)
        lanew = jax.lax.broadcasted_iota(jnp.int32, (_W, 128), 1)
        win = []
        for t in range(_K):
            mk = jnp.max(ck8, axis=1, keepdims=True)
            mi = jnp.min(jnp.where(ck8 == mk, ci8, _BIGF), axis=1,
                         keepdims=True)
            kill = (ck8 == mk) & (ci8 == mi)
            ck8 = jnp.where(kill, _NEG, ck8)
            ci8 = jnp.where(kill, _BIGF, ci8)
            out3 = jnp.where(lanew == t, mk, out3)
            out3 = jnp.where(lanew == _K + t, mi, out3)
            win.append((mk, mi))
        top3_ref[pl.ds(w * _W, _W), :] = out3

        prow = [out_ref[pl.ds(w * _W, _W), 2 * d + t * d:2 * d + (t + 1) * d]
                for t in range(_K)]
        new = []
        for t in range(_K):
            mk, mi = win[t]
            is_loc = mi >= blo
            gmat = ((idxf == mi) & is_loc).astype(jnp.bfloat16)
            acc = jnp.dot(gmat, xbf, preferred_element_type=jnp.float32)
            for s in range(_K):
                f = (~is_loc) & (mk == prevk[s]) & (mi == previ[s])
                acc = acc + f.astype(jnp.float32) * prow[s]
            new.append(acc)
        for t in range(_K):
            out_ref[pl.ds(w * _W, _W), 2 * d + t * d:2 * d + (t + 1) * d] = (
                new[t])
        return carry

    jax.lax.fori_loop(w0, w1 + 1, wbody, jnp.int32(0))

    @pl.when(g == nblocks - 1)
    def _finalize():
        cnt = cnt_ref[:, 0:1]
        out_ref[:, 0:d] = out_ref[:, d:2 * d] / jnp.maximum(cnt, 1.0)


def kernel(x, batch):
    n, d = x.shape
    nb = pl.cdiv(n, _R)
    npad = nb * _R
    batchp = jnp.pad(batch.astype(jnp.int32), (0, npad - n),
                     constant_values=_B).reshape(nb, 1, _R)
    w0s = batchp[:, 0, 0] // _W
    w1s = jnp.minimum(batchp[:, 0, _R - 1], _B - 1) // _W

    out = pl.pallas_call(
        functools.partial(_pool_kernel, nblocks=nb, n=n, d=d),
        grid_spec=pltpu.PrefetchScalarGridSpec(
            num_scalar_prefetch=2,
            grid=(nb,),
            in_specs=[
                pl.BlockSpec((_R, d), lambda g, w0s, w1s: (g, 0)),
                pl.BlockSpec((nb, 1, _R), lambda g, w0s, w1s: (0, 0, 0)),
            ],
            out_specs=pl.BlockSpec((_B, (2 + _K) * d),
                                   lambda g, w0s, w1s: (0, 0)),
            scratch_shapes=[pltpu.VMEM((_B, 128), jnp.float32),
                            pltpu.VMEM((_B, 128), jnp.float32)],
        ),
        out_shape=jax.ShapeDtypeStruct((_B, (2 + _K) * d), jnp.float32),
        interpret=_INTERPRET,
    )(w0s, w1s, x, batchp)
    return out


# speedup vs baseline: 1.1125x; 1.1125x over previous
import functools

import jax
import jax.numpy as jnp
from jax.experimental import pallas as pl
from jax.experimental.pallas import tpu as pltpu

_B = 512
_K = 3
_R = 2048
_W = 32
_NEG = float("-inf")
_BIGF = 1e9


def _pool_kernel(w0ref, w1ref, xref, bref, out_ref, cnt_ref, top3_ref,
                 *, nblocks, n, d):
    g = pl.program_id(0)

    @pl.when(g == 0)
    def _init():
        out_ref[...] = jnp.zeros_like(out_ref)
        cnt_ref[...] = jnp.zeros_like(cnt_ref)
        lane = jax.lax.broadcasted_iota(jnp.int32, (_B, 128), 1)
        top3_ref[...] = jnp.where(lane < _K, _NEG, _BIGF)

    xb = xref[...]
    rowid = g * _R + jax.lax.broadcasted_iota(jnp.int32, (_R, 1), 0)
    xb = jnp.where(rowid < n, xb, 0.0)
    xbf = xb.astype(jnp.bfloat16)
    bt = bref[g]

    kj = jnp.transpose(xb[:, -1:])
    idxf = (g * _R + jax.lax.broadcasted_iota(jnp.int32, (1, _R), 1)
            ).astype(jnp.float32)
    w0 = w0ref[g]
    w1 = w1ref[g]
    wseg = jax.lax.broadcasted_iota(jnp.int32, (_W, 1), 0)
    lane8 = jax.lax.broadcasted_iota(jnp.int32, (_W, 8), 1)
    blo = jnp.float32(g) * _R

    def wbody(w, carry):
        act = bt == (w * _W + wseg)
        out_ref[pl.ds(w * _W, _W), d:2 * d] += jnp.dot(
            act.astype(jnp.bfloat16), xbf,
            preferred_element_type=jnp.float32)
        cnt_ref[pl.ds(w * _W, _W), 0:1] += jnp.sum(
            act.astype(jnp.float32), axis=1, keepdims=True)

        loc = []
        for _ in range(_K):
            ck = jnp.where(act, kj, _NEG)
            m = jnp.max(ck, axis=1, keepdims=True)
            eqm = act & (kj == m)
            im = jnp.min(jnp.where(eqm, idxf, _BIGF), axis=1, keepdims=True)
            act = act & ~(eqm & (idxf == im))
            loc.append((m, im))

        prev = top3_ref[pl.ds(w * _W, _W), :]
        prevk = [prev[:, t:t + 1] for t in range(_K)]
        previ = [prev[:, _K + t:_K + t + 1] for t in range(_K)]
        ck8 = jnp.full((_W, 8), _NEG, jnp.float32)
        ci8 = jnp.full((_W, 8), _BIGF, jnp.float32)
        for t in range(_K):
            ck8 = jnp.where(lane8 == t, loc[t][0], ck8)
            ci8 = jnp.where(lane8 == t, loc[t][1], ci8)
            ck8 = jnp.where(lane8 == _K + t, prevk[t], ck8)
            ci8 = jnp.where(lane8 == _K + t, previ[t], ci8)
        out3 = jnp.full((_W, 128), _BIGF, jnp.float32)
        lanew = jax.lax.broadcasted_iota(jnp.int32, (_W, 128), 1)
        win = []
        for t in range(_K):
            mk = jnp.max(ck8, axis=1, keepdims=True)
            mi = jnp.min(jnp.where(ck8 == mk, ci8, _BIGF), axis=1,
                         keepdims=True)
            kill = (ck8 == mk) & (ci8 == mi)
            ck8 = jnp.where(kill, _NEG, ck8)
            ci8 = jnp.where(kill, _BIGF, ci8)
            out3 = jnp.where(lanew == t, mk, out3)
            out3 = jnp.where(lanew == _K + t, mi, out3)
            win.append((mk, mi))
        top3_ref[pl.ds(w * _W, _W), :] = out3

        prow = [out_ref[pl.ds(w * _W, _W), 2 * d + t * d:2 * d + (t + 1) * d]
                for t in range(_K)]
        new = []
        for t in range(_K):
            mk, mi = win[t]
            is_loc = mi >= blo
            gmat = ((idxf == mi) & is_loc).astype(jnp.bfloat16)
            acc = jnp.dot(gmat, xbf, preferred_element_type=jnp.float32)
            for s in range(_K):
                f = (~is_loc) & (mk == prevk[s]) & (mi == previ[s])
                acc = acc + f.astype(jnp.float32) * prow[s]
            new.append(acc)
        for t in range(_K):
            out_ref[pl.ds(w * _W, _W), 2 * d + t * d:2 * d + (t + 1) * d] = (
                new[t])
        return carry

    jax.lax.fori_loop(w0, w1 + 1, wbody, jnp.int32(0))

    @pl.when(g == nblocks - 1)
    def _finalize():
        cnt = cnt_ref[:, 0:1]
        out_ref[:, 0:d] = out_ref[:, d:2 * d] / jnp.maximum(cnt, 1.0)


def kernel(x, batch):
    n, d = x.shape
    nb = pl.cdiv(n, _R)
    npad = nb * _R
    batchp = jnp.pad(batch.astype(jnp.int32), (0, npad - n),
                     constant_values=_B).reshape(nb, 1, _R)
    w0s = batchp[:, 0, 0] // _W
    w1s = jnp.minimum(batchp[:, 0, _R - 1], _B - 1) // _W

    out = pl.pallas_call(
        functools.partial(_pool_kernel, nblocks=nb, n=n, d=d),
        grid_spec=pltpu.PrefetchScalarGridSpec(
            num_scalar_prefetch=2,
            grid=(nb,),
            in_specs=[
                pl.BlockSpec((_R, d), lambda g, w0s, w1s: (g, 0)),
                pl.BlockSpec((nb, 1, _R), lambda g, w0s, w1s: (0, 0, 0)),
            ],
            out_specs=pl.BlockSpec((_B, (2 + _K) * d),
                                   lambda g, w0s, w1s: (0, 0)),
            scratch_shapes=[pltpu.VMEM((_B, 128), jnp.float32),
                            pltpu.VMEM((_B, 128), jnp.float32)],
        ),
        out_shape=jax.ShapeDtypeStruct((_B, (2 + _K) * d), jnp.float32),
    )(w0s, w1s, x, batchp)
    return out
